# baseline (speedup 1.0000x reference)
import functools

import jax
import jax.numpy as jnp
from jax import lax
from jax.experimental import pallas as pl
from jax.experimental.pallas import tpu as pltpu
from jax.experimental.pallas import tpu_sc as plsc

N = 10000
E = 320000
H = 128
R = 8
NBLK = 4
SB = H // NBLK

NC = 2
NS = 16
NW = NC * NS
LANES = 16

CE = 64
CH = 160
NBUF = 5
EPW = CH * CE
EPAD = NW * EPW
NWCH = NW * CH
ROWS_PER_TILE = 632
NPAD = NS * ROWS_PER_TILE

BLK = 400
GRID = N // BLK


def _fusion_transform_kernel(emb_d, proj_w, proj_b, emb_t, w1, b1, g1, bb1,
                             w2, b2, g2, bb2, raw_d, uw1, ub1, uw2t, ub2,
                             ln_g, ln_b, wbd, out_ref):
    inv = 1.0 / jnp.sqrt(jnp.float32(1.0 + 1e-5))
    d_emb = jnp.dot(emb_d[...], proj_w[...],
                    preferred_element_type=jnp.float32) + proj_b[...]
    h = jnp.dot(emb_t[...], w1[...],
                preferred_element_type=jnp.float32) + b1[...]
    h = h * inv * g1[...] + bb1[...]
    h = jnp.maximum(h, 0.0)
    t = jnp.dot(h, w2[...], preferred_element_type=jnp.float32) + b2[...]
    t = t * inv * g2[...] + bb2[...]
    raw = raw_d[...]
    geo = jnp.sqrt(jnp.sum(raw * raw, axis=-1, keepdims=True))
    u_h = jnp.maximum(geo * uw1[...] + ub1[...], 0.0)
    unc = jax.nn.sigmoid(jnp.sum(u_h * uw2t[...], axis=-1, keepdims=True)
                         + ub2[...])
    x = 0.5 * d_emb + 0.5 * t + unc * t
    mu = jnp.mean(x, axis=-1, keepdims=True)
    var = jnp.mean((x - mu) * (x - mu), axis=-1, keepdims=True)
    x = (x - mu) / jnp.sqrt(var + 1e-5) * ln_g[...] + ln_b[...]
    for r in range(R):
        out_ref[r] = jnp.dot(x, wbd[r], preferred_element_type=jnp.float32)


def _mid_transform_kernel(agg, b0, wbd, out_ref):
    y = jnp.maximum(agg[0] + agg[1] + b0[...], 0.0)
    for r in range(R):
        out_ref[r] = jnp.dot(y, wbd[r], preferred_element_type=jnp.float32)


def _final_kernel(agg, b1, out_ref):
    out_ref[...] = agg[0] + agg[1] + b1[...]


def _row_spec(shape):
    nd = len(shape)
    return pl.BlockSpec(shape, lambda i: (0,) * nd)


def _sc_edge_body(trans_hbm, meta_hbm, out_hbm, *scr):
    cid = lax.axis_index("c")
    tid = lax.axis_index("s")
    wid = cid * NS + tid
    metas = scr[0:NBUF]
    rows = scr[NBUF:2 * NBUF]
    agg_sh = scr[2 * NBUF]
    sems_m = scr[2 * NBUF + 1:2 * NBUF + 1 + NBUF]
    sems_g = scr[2 * NBUF + 1 + NBUF:]
    rows0 = rows[0]

    @pl.loop(0, CE)
    def _zrow(i):
        for j in range(8):
            rows0[i, pl.ds(j * LANES, LANES)] = jnp.zeros((LANES,),
                                                          jnp.float32)

    base = tid * ROWS_PER_TILE
    full = ROWS_PER_TILE // CE
    rem = ROWS_PER_TILE - full * CE
    for k in range(full):
        pltpu.sync_copy(rows0, agg_sh.at[pl.ds(base + k * CE, CE)])
    if rem:
        pltpu.sync_copy(rows0.at[pl.ds(0, rem)],
                        agg_sh.at[pl.ds(base + full * CE, rem)])

    plsc.subcore_barrier()

    for o in range(NBUF):
        pltpu.async_copy(meta_hbm.at[wid, o], metas[o], sems_m[o])
    for o in range(NBUF - 1):
        pltpu.make_async_copy(meta_hbm.at[wid, o], metas[o],
                              sems_m[o]).wait()
        pltpu.async_copy(trans_hbm.at[metas[o].at[0]], rows[o], sems_g[o])

    @pl.loop(0, CH, step=NBUF)
    def _chunk4(cb):
        for o in range(NBUF):
            c = cb + o
            b = o
            b3 = (o + NBUF - 1) % NBUF
            pltpu.make_async_copy(trans_hbm.at[metas[b].at[0]], rows[b],
                                  sems_g[b]).wait()

            @pl.loop(0, CE // LANES)
            def _scale(g, b=b):
                nvs = plsc.bitcast(metas[b][2, pl.ds(g * LANES, LANES)],
                                   jnp.float32)
                for k in range(LANES):
                    e = g * LANES + k
                    nv = nvs[k]
                    for j in range(8):
                        sl = pl.ds(j * LANES, LANES)
                        rows[b][e, sl] = rows[b][e, sl] * nv

            pltpu.sync_copy(rows[b], agg_sh.at[metas[b].at[1]], add=True)

            @pl.when(c + NBUF < CH)
            def _(b=b, c=c):
                pltpu.async_copy(meta_hbm.at[wid, c + NBUF], metas[b],
                                 sems_m[b])

            @pl.when(c + NBUF - 1 < CH)
            def _(b3=b3, c=c):
                pltpu.make_async_copy(meta_hbm.at[wid, 0], metas[b3],
                                      sems_m[b3]).wait()
                pltpu.async_copy(trans_hbm.at[metas[b3].at[0]], rows[b3],
                                 sems_g[b3])

    plsc.subcore_barrier()

    pltpu.sync_copy(agg_sh.at[pl.ds(base, ROWS_PER_TILE)],
                    out_hbm.at[cid, pl.ds(base, ROWS_PER_TILE)])


_sc_edge_pass = functools.partial(
    pl.kernel,
    out_type=jax.ShapeDtypeStruct((NC, NPAD, H), jnp.float32),
    mesh=plsc.VectorSubcoreMesh(core_axis_name="c", subcore_axis_name="s",
                                num_cores=NC, num_subcores=NS),
    compiler_params=pltpu.CompilerParams(needs_layout_passes=False),
    scratch_types=(
        [pltpu.VMEM((3, CE), jnp.int32) for _ in range(NBUF)]
        + [pltpu.VMEM((CE, H), jnp.float32) for _ in range(NBUF)]
        + [pltpu.VMEM_SHARED((NPAD, H), jnp.float32)]
        + [pltpu.SemaphoreType.DMA] * (2 * NBUF)
    ),
)(_sc_edge_body)


def _meta_pack_kernel(g2, rel, nrm, out_ref):
    out_ref[:, 0, :] = g2[0] + rel[...] * N
    out_ref[:, 1, :] = g2[1]
    out_ref[:, 2, :] = lax.bitcast_convert_type(nrm[...], jnp.int32)


def _block_diag(w):
    out = jnp.zeros((R, H, H), jnp.float32)
    for b in range(NBLK):
        out = out.at[:, b * SB:(b + 1) * SB, b * SB:(b + 1) * SB].set(w[:, b])
    return out


def kernel(graph, node_ids, rel_ids, norm, params):
    p = params
    del node_ids

    rel = rel_ids.astype(jnp.int32)

    pad = EPAD - E
    spread = jnp.arange(pad, dtype=jnp.int32) % jnp.int32(N)
    g2w = jnp.concatenate(
        [graph.astype(jnp.int32),
         jnp.broadcast_to(spread, (2, pad))], axis=1).reshape(2, NWCH, CE)
    relw = jnp.pad(rel, (0, pad)).reshape(NWCH, CE)
    nrmw = jnp.pad(norm, ((0, pad), (0, 0))).reshape(NWCH, CE)

    MB = 256
    meta = pl.pallas_call(
        _meta_pack_kernel,
        grid=(NWCH // MB,),
        in_specs=[
            pl.BlockSpec((2, MB, CE), lambda i: (0, i, 0)),
            pl.BlockSpec((MB, CE), lambda i: (i, 0)),
            pl.BlockSpec((MB, CE), lambda i: (i, 0)),
        ],
        out_specs=pl.BlockSpec((MB, 3, CE), lambda i: (i, 0, 0)),
        out_shape=jax.ShapeDtypeStruct((NWCH, 3, CE), jnp.int32),
    )(g2w, relw, nrmw).reshape(NW, CH, 3, CE)

    wbd0 = _block_diag(p['rgcn0_W'])
    wbd1 = _block_diag(p['rgcn1_W'])

    row2 = lambda v: v.reshape(1, -1)

    trans0 = pl.pallas_call(
        _fusion_transform_kernel,
        grid=(GRID,),
        in_specs=[
            pl.BlockSpec((BLK, H), lambda i: (i, 0)),
            _row_spec((H, H)),
            _row_spec((1, H)),
            pl.BlockSpec((BLK, H), lambda i: (i, 0)),
            _row_spec((H, 2 * H)),
            _row_spec((1, 2 * H)),
            _row_spec((1, 2 * H)),
            _row_spec((1, 2 * H)),
            _row_spec((2 * H, H)),
            _row_spec((1, H)),
            _row_spec((1, H)),
            _row_spec((1, H)),
            pl.BlockSpec((BLK, H), lambda i: (i, 0)),
            _row_spec((1, 16)),
            _row_spec((1, 16)),
            _row_spec((1, 16)),
            _row_spec((1, 1)),
            _row_spec((1, H)),
            _row_spec((1, H)),
            _row_spec((R, H, H)),
        ],
        out_specs=pl.BlockSpec((R, BLK, H), lambda i: (0, i, 0)),
        out_shape=jax.ShapeDtypeStruct((R, N, H), jnp.float32),
    )(p['emb_domain'], p['proj_W'], row2(p['proj_b']), p['emb_text'],
      p['enc_W1'], row2(p['enc_b1']), row2(p['bn1_g']), row2(p['bn1_b']),
      p['enc_W2'], row2(p['enc_b2']), row2(p['bn2_g']), row2(p['bn2_b']),
      p['raw_domain'], row2(p['u_W1'].reshape(-1)), row2(p['u_b1']),
      row2(p['u_W2'].reshape(-1)), p['u_b2'].reshape(1, 1),
      row2(p['ln_g']), row2(p['ln_b']), wbd0)

    agg0 = _sc_edge_pass(trans0.reshape(R * N, H), meta)

    trans1 = pl.pallas_call(
        _mid_transform_kernel,
        grid=(GRID,),
        in_specs=[
            pl.BlockSpec((NC, BLK, H), lambda i: (0, i, 0)),
            _row_spec((1, H)),
            _row_spec((R, H, H)),
        ],
        out_specs=pl.BlockSpec((R, BLK, H), lambda i: (0, i, 0)),
        out_shape=jax.ShapeDtypeStruct((R, N, H), jnp.float32),
    )(agg0, row2(p['rgcn0_b']), wbd1)

    agg1 = _sc_edge_pass(trans1.reshape(R * N, H), meta)

    out = pl.pallas_call(
        _final_kernel,
        grid=(GRID,),
        in_specs=[
            pl.BlockSpec((NC, BLK, H), lambda i: (0, i, 0)),
            _row_spec((1, H)),
        ],
        out_specs=pl.BlockSpec((BLK, H), lambda i: (i, 0)),
        out_shape=jax.ShapeDtypeStruct((N, H), jnp.float32),
    )(agg1, row2(p['rgcn1_b']))

    return out


# speedup vs baseline: 1.1133x; 1.1133x over previous
import functools

import jax
import jax.numpy as jnp
from jax import lax
from jax.experimental import pallas as pl
from jax.experimental.pallas import tpu as pltpu
from jax.experimental.pallas import tpu_sc as plsc

N = 10000
E = 320000
H = 128
R = 8
NBLK = 4
SB = H // NBLK

NC = 2
NS = 16
NW = NC * NS
LANES = 16

CE = 64
CH = 160
NBUF = 4
EPW = CH * CE
EPAD = NW * EPW
NWCH = NW * CH
ROWS_PER_TILE = 632
NPAD = NS * ROWS_PER_TILE

BLK = 400
GRID = N // BLK


def _fusion_transform_kernel(emb_d, proj_w, proj_b, emb_t, w1, b1, g1, bb1,
                             w2, b2, g2, bb2, raw_d, uw1, ub1, uw2t, ub2,
                             ln_g, ln_b, wbd, out_ref):
    inv = 1.0 / jnp.sqrt(jnp.float32(1.0 + 1e-5))
    d_emb = jnp.dot(emb_d[...], proj_w[...],
                    preferred_element_type=jnp.float32) + proj_b[...]
    h = jnp.dot(emb_t[...], w1[...],
                preferred_element_type=jnp.float32) + b1[...]
    h = h * inv * g1[...] + bb1[...]
    h = jnp.maximum(h, 0.0)
    t = jnp.dot(h, w2[...], preferred_element_type=jnp.float32) + b2[...]
    t = t * inv * g2[...] + bb2[...]
    raw = raw_d[...]
    geo = jnp.sqrt(jnp.sum(raw * raw, axis=-1, keepdims=True))
    u_h = jnp.maximum(geo * uw1[...] + ub1[...], 0.0)
    unc = jax.nn.sigmoid(jnp.sum(u_h * uw2t[...], axis=-1, keepdims=True)
                         + ub2[...])
    x = 0.5 * d_emb + 0.5 * t + unc * t
    mu = jnp.mean(x, axis=-1, keepdims=True)
    var = jnp.mean((x - mu) * (x - mu), axis=-1, keepdims=True)
    x = (x - mu) / jnp.sqrt(var + 1e-5) * ln_g[...] + ln_b[...]
    for r in range(R):
        out_ref[r] = jnp.dot(x, wbd[r], preferred_element_type=jnp.float32)


def _mid_transform_kernel(agg, b0, wbd, out_ref):
    y = jnp.maximum(agg[0] + agg[1] + b0[...], 0.0)
    for r in range(R):
        out_ref[r] = jnp.dot(y, wbd[r], preferred_element_type=jnp.float32)


def _final_kernel(agg, b1, out_ref):
    out_ref[...] = agg[0] + agg[1] + b1[...]


def _row_spec(shape):
    nd = len(shape)
    return pl.BlockSpec(shape, lambda i: (0,) * nd)


def _sc_edge_body(trans_hbm, meta_hbm, out_hbm, *scr):
    cid = lax.axis_index("c")
    tid = lax.axis_index("s")
    wid = cid * NS + tid
    metas = scr[0:NBUF]
    rows = scr[NBUF:2 * NBUF]
    agg_sh = scr[2 * NBUF]
    sems_m = scr[2 * NBUF + 1:2 * NBUF + 1 + NBUF]
    sems_g = scr[2 * NBUF + 1 + NBUF:]
    rows0 = rows[0]

    @pl.loop(0, CE)
    def _zrow(i):
        for j in range(8):
            rows0[i, pl.ds(j * LANES, LANES)] = jnp.zeros((LANES,),
                                                          jnp.float32)

    base = tid * ROWS_PER_TILE
    full = ROWS_PER_TILE // CE
    rem = ROWS_PER_TILE - full * CE
    for k in range(full):
        pltpu.sync_copy(rows0, agg_sh.at[pl.ds(base + k * CE, CE)])
    if rem:
        pltpu.sync_copy(rows0.at[pl.ds(0, rem)],
                        agg_sh.at[pl.ds(base + full * CE, rem)])

    plsc.subcore_barrier()

    for o in range(NBUF):
        pltpu.async_copy(meta_hbm.at[wid, o], metas[o], sems_m[o])
    for o in range(NBUF - 1):
        pltpu.make_async_copy(meta_hbm.at[wid, o], metas[o],
                              sems_m[o]).wait()
        pltpu.async_copy(trans_hbm.at[metas[o].at[0]], rows[o], sems_g[o])

    @pl.loop(0, CH, step=NBUF)
    def _chunk4(cb):
        for o in range(NBUF):
            c = cb + o
            b = o
            b3 = (o + NBUF - 1) % NBUF
            pltpu.make_async_copy(trans_hbm.at[metas[b].at[0]], rows[b],
                                  sems_g[b]).wait()

            @pl.loop(0, CE // LANES)
            def _scale(g, b=b):
                nvs = plsc.bitcast(metas[b][2, pl.ds(g * LANES, LANES)],
                                   jnp.float32)
                for k in range(LANES):
                    e = g * LANES + k
                    nv = nvs[k]
                    for j in range(8):
                        sl = pl.ds(j * LANES, LANES)
                        rows[b][e, sl] = rows[b][e, sl] * nv

            pltpu.sync_copy(rows[b], agg_sh.at[metas[b].at[1]], add=True)

            @pl.when(c + NBUF < CH)
            def _(b=b, c=c):
                pltpu.async_copy(meta_hbm.at[wid, c + NBUF], metas[b],
                                 sems_m[b])

            @pl.when(c + NBUF - 1 < CH)
            def _(b3=b3, c=c):
                pltpu.make_async_copy(meta_hbm.at[wid, 0], metas[b3],
                                      sems_m[b3]).wait()
                pltpu.async_copy(trans_hbm.at[metas[b3].at[0]], rows[b3],
                                 sems_g[b3])

    plsc.subcore_barrier()

    pltpu.sync_copy(agg_sh.at[pl.ds(base, ROWS_PER_TILE)],
                    out_hbm.at[cid, pl.ds(base, ROWS_PER_TILE)])


_sc_edge_pass = functools.partial(
    pl.kernel,
    out_type=jax.ShapeDtypeStruct((NC, NPAD, H), jnp.float32),
    mesh=plsc.VectorSubcoreMesh(core_axis_name="c", subcore_axis_name="s",
                                num_cores=NC, num_subcores=NS),
    compiler_params=pltpu.CompilerParams(needs_layout_passes=False),
    scratch_types=(
        [pltpu.VMEM((3, CE), jnp.int32) for _ in range(NBUF)]
        + [pltpu.VMEM((CE, H), jnp.float32) for _ in range(NBUF)]
        + [pltpu.VMEM_SHARED((NPAD, H), jnp.float32)]
        + [pltpu.SemaphoreType.DMA] * (2 * NBUF)
    ),
)(_sc_edge_body)


def _meta_pack_kernel(g2, rel, nrm, out_ref):
    out_ref[:, 0, :] = g2[0] + rel[...] * N
    out_ref[:, 1, :] = g2[1]
    out_ref[:, 2, :] = lax.bitcast_convert_type(nrm[...], jnp.int32)


def _block_diag(w):
    out = jnp.zeros((R, H, H), jnp.float32)
    for b in range(NBLK):
        out = out.at[:, b * SB:(b + 1) * SB, b * SB:(b + 1) * SB].set(w[:, b])
    return out


def kernel(graph, node_ids, rel_ids, norm, params):
    p = params
    del node_ids

    rel = rel_ids.astype(jnp.int32)

    pad = EPAD - E
    spread = jnp.arange(pad, dtype=jnp.int32) % jnp.int32(N)
    g2w = jnp.concatenate(
        [graph.astype(jnp.int32),
         jnp.broadcast_to(spread, (2, pad))], axis=1).reshape(2, NWCH, CE)
    relw = jnp.pad(rel, (0, pad)).reshape(NWCH, CE)
    nrmw = jnp.pad(norm, ((0, pad), (0, 0))).reshape(NWCH, CE)

    MB = 256
    meta = pl.pallas_call(
        _meta_pack_kernel,
        grid=(NWCH // MB,),
        in_specs=[
            pl.BlockSpec((2, MB, CE), lambda i: (0, i, 0)),
            pl.BlockSpec((MB, CE), lambda i: (i, 0)),
            pl.BlockSpec((MB, CE), lambda i: (i, 0)),
        ],
        out_specs=pl.BlockSpec((MB, 3, CE), lambda i: (i, 0, 0)),
        out_shape=jax.ShapeDtypeStruct((NWCH, 3, CE), jnp.int32),
    )(g2w, relw, nrmw).reshape(NW, CH, 3, CE)

    wbd0 = _block_diag(p['rgcn0_W'])
    wbd1 = _block_diag(p['rgcn1_W'])

    row2 = lambda v: v.reshape(1, -1)

    trans0 = pl.pallas_call(
        _fusion_transform_kernel,
        grid=(GRID,),
        in_specs=[
            pl.BlockSpec((BLK, H), lambda i: (i, 0)),
            _row_spec((H, H)),
            _row_spec((1, H)),
            pl.BlockSpec((BLK, H), lambda i: (i, 0)),
            _row_spec((H, 2 * H)),
            _row_spec((1, 2 * H)),
            _row_spec((1, 2 * H)),
            _row_spec((1, 2 * H)),
            _row_spec((2 * H, H)),
            _row_spec((1, H)),
            _row_spec((1, H)),
            _row_spec((1, H)),
            pl.BlockSpec((BLK, H), lambda i: (i, 0)),
            _row_spec((1, 16)),
            _row_spec((1, 16)),
            _row_spec((1, 16)),
            _row_spec((1, 1)),
            _row_spec((1, H)),
            _row_spec((1, H)),
            _row_spec((R, H, H)),
        ],
        out_specs=pl.BlockSpec((R, BLK, H), lambda i: (0, i, 0)),
        out_shape=jax.ShapeDtypeStruct((R, N, H), jnp.float32),
    )(p['emb_domain'], p['proj_W'], row2(p['proj_b']), p['emb_text'],
      p['enc_W1'], row2(p['enc_b1']), row2(p['bn1_g']), row2(p['bn1_b']),
      p['enc_W2'], row2(p['enc_b2']), row2(p['bn2_g']), row2(p['bn2_b']),
      p['raw_domain'], row2(p['u_W1'].reshape(-1)), row2(p['u_b1']),
      row2(p['u_W2'].reshape(-1)), p['u_b2'].reshape(1, 1),
      row2(p['ln_g']), row2(p['ln_b']), wbd0)

    agg0 = _sc_edge_pass(trans0.reshape(R * N, H), meta)

    trans1 = pl.pallas_call(
        _mid_transform_kernel,
        grid=(GRID,),
        in_specs=[
            pl.BlockSpec((NC, BLK, H), lambda i: (0, i, 0)),
            _row_spec((1, H)),
            _row_spec((R, H, H)),
        ],
        out_specs=pl.BlockSpec((R, BLK, H), lambda i: (0, i, 0)),
        out_shape=jax.ShapeDtypeStruct((R, N, H), jnp.float32),
    )(agg0, row2(p['rgcn0_b']), wbd1)

    agg1 = _sc_edge_pass(trans1.reshape(R * N, H), meta)

    out = pl.pallas_call(
        _final_kernel,
        grid=(GRID,),
        in_specs=[
            pl.BlockSpec((NC, BLK, H), lambda i: (0, i, 0)),
            _row_spec((1, H)),
        ],
        out_specs=pl.BlockSpec((BLK, H), lambda i: (i, 0)),
        out_shape=jax.ShapeDtypeStruct((N, H), jnp.float32),
    )(agg1, row2(p['rgcn1_b']))

    return out


# speedup vs baseline: 1.1147x; 1.0013x over previous
import functools

import jax
import jax.numpy as jnp
from jax import lax
from jax.experimental import pallas as pl
from jax.experimental.pallas import tpu as pltpu
from jax.experimental.pallas import tpu_sc as plsc

N = 10000
E = 320000
H = 128
R = 8
NBLK = 4
SB = H // NBLK

NC = 2
NS = 16
NW = NC * NS
LANES = 16

CE = 64
CH = 160
NBUF = 4
EPW = CH * CE
EPAD = NW * EPW
NWCH = NW * CH
ROWS_PER_TILE = 632
NPAD = NS * ROWS_PER_TILE

BLK = 400
GRID = N // BLK


def _fusion_transform_kernel(emb_d, proj_w, proj_b, emb_t, w1, b1, g1, bb1,
                             w2, b2, g2, bb2, raw_d, uw1, ub1, uw2t, ub2,
                             ln_g, ln_b, wbd, out_ref):
    inv = 1.0 / jnp.sqrt(jnp.float32(1.0 + 1e-5))
    d_emb = jnp.dot(emb_d[...], proj_w[...],
                    preferred_element_type=jnp.float32) + proj_b[...]
    h = jnp.dot(emb_t[...], w1[...],
                preferred_element_type=jnp.float32) + b1[...]
    h = h * inv * g1[...] + bb1[...]
    h = jnp.maximum(h, 0.0)
    t = jnp.dot(h, w2[...], preferred_element_type=jnp.float32) + b2[...]
    t = t * inv * g2[...] + bb2[...]
    raw = raw_d[...]
    geo = jnp.sqrt(jnp.sum(raw * raw, axis=-1, keepdims=True))
    u_h = jnp.maximum(geo * uw1[...] + ub1[...], 0.0)
    unc = jax.nn.sigmoid(jnp.sum(u_h * uw2t[...], axis=-1, keepdims=True)
                         + ub2[...])
    x = 0.5 * d_emb + 0.5 * t + unc * t
    mu = jnp.mean(x, axis=-1, keepdims=True)
    var = jnp.mean((x - mu) * (x - mu), axis=-1, keepdims=True)
    x = (x - mu) / jnp.sqrt(var + 1e-5) * ln_g[...] + ln_b[...]
    for r in range(R):
        out_ref[r] = jnp.dot(x, wbd[r], preferred_element_type=jnp.float32)


def _mid_transform_kernel(agg, b0, wbd, out_ref):
    y = jnp.maximum(agg[0] + agg[1] + b0[...], 0.0)
    for r in range(R):
        out_ref[r] = jnp.dot(y, wbd[r], preferred_element_type=jnp.float32)


def _final_kernel(agg, b1, out_ref):
    out_ref[...] = agg[0] + agg[1] + b1[...]


def _row_spec(shape):
    nd = len(shape)
    return pl.BlockSpec(shape, lambda i: (0,) * nd)


def _sc_edge_body(trans_hbm, meta_hbm, out_hbm, *scr):
    cid = lax.axis_index("c")
    tid = lax.axis_index("s")
    wid = cid * NS + tid
    metas = scr[0:NBUF]
    rows = scr[NBUF:2 * NBUF]
    agg_sh = scr[2 * NBUF]
    sems_m = scr[2 * NBUF + 1:2 * NBUF + 1 + NBUF]
    sems_g = scr[2 * NBUF + 1 + NBUF:]
    rows0 = rows[0]

    @pl.loop(0, CE)
    def _zrow(i):
        for j in range(8):
            rows0[i, pl.ds(j * LANES, LANES)] = jnp.zeros((LANES,),
                                                          jnp.float32)

    base = tid * ROWS_PER_TILE
    full = ROWS_PER_TILE // CE
    rem = ROWS_PER_TILE - full * CE
    for k in range(full):
        pltpu.sync_copy(rows0, agg_sh.at[pl.ds(base + k * CE, CE)])
    if rem:
        pltpu.sync_copy(rows0.at[pl.ds(0, rem)],
                        agg_sh.at[pl.ds(base + full * CE, rem)])

    plsc.subcore_barrier()

    for o in range(NBUF):
        pltpu.async_copy(meta_hbm.at[wid, o], metas[o], sems_m[o])
    for o in range(NBUF - 1):
        pltpu.make_async_copy(meta_hbm.at[wid, o], metas[o],
                              sems_m[o]).wait()
        pltpu.async_copy(trans_hbm.at[metas[o].at[0]], rows[o], sems_g[o])

    @pl.loop(0, CH, step=NBUF)
    def _chunk4(cb):
        for o in range(NBUF):
            c = cb + o
            b = o
            b3 = (o + NBUF - 1) % NBUF
            pltpu.make_async_copy(trans_hbm.at[metas[b].at[0]], rows[b],
                                  sems_g[b]).wait()

            @pl.loop(0, CE // LANES)
            def _scale(g, b=b):
                nvs = plsc.bitcast(metas[b][2, pl.ds(g * LANES, LANES)],
                                   jnp.float32)
                for k in range(LANES):
                    e = g * LANES + k
                    nv = nvs[k]
                    for j in range(8):
                        sl = pl.ds(j * LANES, LANES)
                        rows[b][e, sl] = rows[b][e, sl] * nv

            pltpu.sync_copy(rows[b], agg_sh.at[metas[b].at[1]], add=True)

            @pl.when(c + NBUF < CH)
            def _(b=b, c=c):
                pltpu.async_copy(meta_hbm.at[wid, c + NBUF], metas[b],
                                 sems_m[b])

            @pl.when(c + NBUF - 1 < CH)
            def _(b3=b3, c=c):
                pltpu.make_async_copy(meta_hbm.at[wid, 0], metas[b3],
                                      sems_m[b3]).wait()
                pltpu.async_copy(trans_hbm.at[metas[b3].at[0]], rows[b3],
                                 sems_g[b3])

    plsc.subcore_barrier()

    pltpu.sync_copy(agg_sh.at[pl.ds(base, ROWS_PER_TILE)],
                    out_hbm.at[cid, pl.ds(base, ROWS_PER_TILE)])


_sc_edge_pass = functools.partial(
    pl.kernel,
    out_type=jax.ShapeDtypeStruct((NC, NPAD, H), jnp.float32),
    mesh=plsc.VectorSubcoreMesh(core_axis_name="c", subcore_axis_name="s",
                                num_cores=NC, num_subcores=NS),
    compiler_params=pltpu.CompilerParams(needs_layout_passes=False),
    scratch_types=(
        [pltpu.VMEM((3, CE), jnp.int32) for _ in range(NBUF)]
        + [pltpu.VMEM((CE, H), jnp.float32) for _ in range(NBUF)]
        + [pltpu.VMEM_SHARED((NPAD, H), jnp.float32)]
        + [pltpu.SemaphoreType.DMA] * (2 * NBUF)
    ),
)(_sc_edge_body)


def _meta_pack_kernel(g2, rel, nrm, out_ref):
    out_ref[:, 0, :] = g2[0] + rel[...] * N
    out_ref[:, 1, :] = g2[1]
    out_ref[:, 2, :] = lax.bitcast_convert_type(nrm[...], jnp.int32)


def _block_diag(w):
    out = jnp.zeros((R, H, H), jnp.float32)
    for b in range(NBLK):
        out = out.at[:, b * SB:(b + 1) * SB, b * SB:(b + 1) * SB].set(w[:, b])
    return out


def kernel(graph, node_ids, rel_ids, norm, params):
    p = params
    del node_ids

    rel = rel_ids.astype(jnp.int32)

    pad = EPAD - E
    spread = jnp.arange(pad, dtype=jnp.int32) % jnp.int32(N)
    g2w = jnp.concatenate(
        [graph.astype(jnp.int32),
         jnp.broadcast_to(spread, (2, pad))], axis=1).reshape(2, NWCH, CE)
    relw = jnp.pad(rel, (0, pad)).reshape(NWCH, CE)
    nrmw = jnp.pad(norm.reshape(1, E), ((0, 0), (0, pad))).reshape(NWCH, CE)

    MB = 256
    meta = pl.pallas_call(
        _meta_pack_kernel,
        grid=(NWCH // MB,),
        in_specs=[
            pl.BlockSpec((2, MB, CE), lambda i: (0, i, 0)),
            pl.BlockSpec((MB, CE), lambda i: (i, 0)),
            pl.BlockSpec((MB, CE), lambda i: (i, 0)),
        ],
        out_specs=pl.BlockSpec((MB, 3, CE), lambda i: (i, 0, 0)),
        out_shape=jax.ShapeDtypeStruct((NWCH, 3, CE), jnp.int32),
    )(g2w, relw, nrmw).reshape(NW, CH, 3, CE)

    wbd0 = _block_diag(p['rgcn0_W'])
    wbd1 = _block_diag(p['rgcn1_W'])

    row2 = lambda v: v.reshape(1, -1)

    trans0 = pl.pallas_call(
        _fusion_transform_kernel,
        grid=(GRID,),
        in_specs=[
            pl.BlockSpec((BLK, H), lambda i: (i, 0)),
            _row_spec((H, H)),
            _row_spec((1, H)),
            pl.BlockSpec((BLK, H), lambda i: (i, 0)),
            _row_spec((H, 2 * H)),
            _row_spec((1, 2 * H)),
            _row_spec((1, 2 * H)),
            _row_spec((1, 2 * H)),
            _row_spec((2 * H, H)),
            _row_spec((1, H)),
            _row_spec((1, H)),
            _row_spec((1, H)),
            pl.BlockSpec((BLK, H), lambda i: (i, 0)),
            _row_spec((1, 16)),
            _row_spec((1, 16)),
            _row_spec((1, 16)),
            _row_spec((1, 1)),
            _row_spec((1, H)),
            _row_spec((1, H)),
            _row_spec((R, H, H)),
        ],
        out_specs=pl.BlockSpec((R, BLK, H), lambda i: (0, i, 0)),
        out_shape=jax.ShapeDtypeStruct((R, N, H), jnp.float32),
    )(p['emb_domain'], p['proj_W'], row2(p['proj_b']), p['emb_text'],
      p['enc_W1'], row2(p['enc_b1']), row2(p['bn1_g']), row2(p['bn1_b']),
      p['enc_W2'], row2(p['enc_b2']), row2(p['bn2_g']), row2(p['bn2_b']),
      p['raw_domain'], row2(p['u_W1'].reshape(-1)), row2(p['u_b1']),
      row2(p['u_W2'].reshape(-1)), p['u_b2'].reshape(1, 1),
      row2(p['ln_g']), row2(p['ln_b']), wbd0)

    agg0 = _sc_edge_pass(trans0.reshape(R * N, H), meta)

    trans1 = pl.pallas_call(
        _mid_transform_kernel,
        grid=(GRID,),
        in_specs=[
            pl.BlockSpec((NC, BLK, H), lambda i: (0, i, 0)),
            _row_spec((1, H)),
            _row_spec((R, H, H)),
        ],
        out_specs=pl.BlockSpec((R, BLK, H), lambda i: (0, i, 0)),
        out_shape=jax.ShapeDtypeStruct((R, N, H), jnp.float32),
    )(agg0, row2(p['rgcn0_b']), wbd1)

    agg1 = _sc_edge_pass(trans1.reshape(R * N, H), meta)

    out = pl.pallas_call(
        _final_kernel,
        grid=(GRID,),
        in_specs=[
            pl.BlockSpec((NC, BLK, H), lambda i: (0, i, 0)),
            _row_spec((1, H)),
        ],
        out_specs=pl.BlockSpec((BLK, H), lambda i: (i, 0)),
        out_shape=jax.ShapeDtypeStruct((N, H), jnp.float32),
    )(agg1, row2(p['rgcn1_b']))

    return out
